# Initial kernel scaffold; baseline (speedup 1.0000x reference)
import jax
import jax.numpy as jnp
from jax.experimental import pallas as pl


def kernel(features, edge_index, W1, b1, W2, b2, W3, b3, W4, b4, Wres, bres):
    raise NotImplementedError("write your pallas kernel here")


# speedup vs baseline: 1.7255x; 1.7255x over previous
import jax
import jax.numpy as jnp
from jax import lax
from jax.experimental import pallas as pl
from jax.experimental.pallas import tpu as pltpu
from jax.experimental.pallas import tpu_sc as plsc

D = 128
K = 80
NC = 2
NS = 16
NW = NC * NS
L = 16
NP = 10240
BLK = 1024


def _mm_body(x_ref, w_ref, b_ref, h1_ref, h23_ref, h4_ref, hr_ref):
    h = jnp.dot(x_ref[...], w_ref[...], preferred_element_type=jnp.float32)
    h = h + b_ref[...]
    h1_ref[...] = h[:, 0:D]
    h23_ref[...] = h[:, D:3 * D]
    h4_ref[...] = h[:, 3 * D:4 * D]
    hr_ref[...] = h[:, 4 * D:5 * D]


def _matmuls(x, wt, bc):
    grid = NP // BLK
    return pl.pallas_call(
        _mm_body,
        grid=(grid,),
        in_specs=[
            pl.BlockSpec((BLK, D), lambda i: (i, 0)),
            pl.BlockSpec((D, 5 * D), lambda i: (0, 0)),
            pl.BlockSpec((1, 5 * D), lambda i: (0, 0)),
        ],
        out_specs=[
            pl.BlockSpec((BLK, D), lambda i: (i, 0)),
            pl.BlockSpec((BLK, 2 * D), lambda i: (i, 0)),
            pl.BlockSpec((BLK, D), lambda i: (i, 0)),
            pl.BlockSpec((BLK, D), lambda i: (i, 0)),
        ],
        out_shape=[
            jax.ShapeDtypeStruct((NP, D), jnp.float32),
            jax.ShapeDtypeStruct((NP, 2 * D), jnp.float32),
            jax.ShapeDtypeStruct((NP, D), jnp.float32),
            jax.ShapeDtypeStruct((NP, D), jnp.float32),
        ],
    )(x, wt, bc)


def _sc_edge_kernel(e):
    nchunks = e // K
    per_core = nchunks // NC
    per_sub = per_core // NS
    rps = NP // NS
    mesh = plsc.VectorSubcoreMesh(core_axis_name="c", subcore_axis_name="s")

    def body(h23_hbm, h4_hbm, src_hbm, dst_hbm, acc_out, cnt_out,
             src_v, dst_v, g23, g4, hist, acc_sh, sem23, sem4):
        cid = lax.axis_index("c")
        sid = lax.axis_index("s")
        wid = sid * NC + cid

        zero = jnp.zeros((L,), jnp.float32)

        def zfill(r, _):
            for j in range(D // L):
                g4[r, pl.ds(j * L, L)] = zero
            return 0
        lax.fori_loop(0, K, zfill, 0, unroll=False)

        def hfill(r, _):
            hist[pl.ds(r * L, L)] = zero
            return 0
        lax.fori_loop(0, NP // L, hfill, 0, unroll=False)

        r0 = pl.multiple_of(sid * rps, 8)
        for t in range(rps // K):
            pltpu.sync_copy(g4, acc_sh.at[pl.ds(r0 + t * K, K)])

        plsc.subcore_barrier()

        def chunk_body(i, _):
            c = cid * per_core + sid + i * NS
            base = c * K
            pltpu.sync_copy(src_hbm.at[pl.ds(base, K)], src_v)
            pltpu.sync_copy(dst_hbm.at[pl.ds(base, K)], dst_v)
            cp23 = pltpu.async_copy(h23_hbm.at[src_v], g23, sem23)
            cp4 = pltpu.async_copy(h4_hbm.at[dst_v], g4, sem4)

            for j in range(K // L):
                idxv = dst_v[pl.ds(j * L, L)]
                cnt, lastm = plsc.scan_count(idxv)
                old = plsc.load_gather(hist, [idxv], mask=lastm)
                plsc.store_scatter(
                    hist, [idxv], old + cnt.astype(jnp.float32),
                    mask=lastm)

            cp23.wait()
            cp4.wait()

            def row_body(r, _):
                for j in range(D // L):
                    sl = pl.ds(j * L, L)
                    x2 = g23[r, sl]
                    x3 = g23[r, pl.ds(D + j * L, L)]
                    x4 = g4[r, sl]
                    g4[r, sl] = x2 / (1.0 + jnp.exp(-(x3 + x4)))
                return 0
            lax.fori_loop(0, K, row_body, 0, unroll=False)

            pltpu.sync_copy(g4, acc_sh.at[dst_v], add=True)
            return 0

        lax.fori_loop(0, per_sub, chunk_body, 0, unroll=False)

        plsc.subcore_barrier()

        for t in range(rps // K):
            row = r0 + t * K
            pltpu.sync_copy(acc_sh.at[pl.ds(row, K)], g4)
            pltpu.sync_copy(g4, acc_out.at[cid, pl.ds(row, K)])

        pltpu.sync_copy(hist, cnt_out.at[pl.ds(wid * NP, NP)])

    return pl.kernel(
        body,
        out_type=[
            jax.ShapeDtypeStruct((NC, NP, D), jnp.float32),
            jax.ShapeDtypeStruct((NW * NP,), jnp.float32),
        ],
        mesh=mesh,
        compiler_params=pltpu.CompilerParams(needs_layout_passes=False),
        scratch_types=[
            pltpu.VMEM((K,), jnp.int32),
            pltpu.VMEM((K,), jnp.int32),
            pltpu.VMEM((K, 2 * D), jnp.float32),
            pltpu.VMEM((K, D), jnp.float32),
            pltpu.VMEM((NP,), jnp.float32),
            pltpu.VMEM_SHARED((NP, D), jnp.float32),
            pltpu.SemaphoreType.DMA,
            pltpu.SemaphoreType.DMA,
        ],
    )


def _comb_body(h1_ref, hr_ref, acc_ref, cnt_ref, o_ref):
    a = acc_ref[0] + acc_ref[1]
    c = jnp.sum(cnt_ref[...], axis=0)[:, None]
    o_ref[...] = (h1_ref[...] + a) / jnp.maximum(c, 1.0) + hr_ref[...]


def _combine(h1, hres, acc, cnt):
    grid = NP // BLK
    return pl.pallas_call(
        _comb_body,
        grid=(grid,),
        in_specs=[
            pl.BlockSpec((BLK, D), lambda i: (i, 0)),
            pl.BlockSpec((BLK, D), lambda i: (i, 0)),
            pl.BlockSpec((NC, BLK, D), lambda i: (0, i, 0)),
            pl.BlockSpec((NW, BLK), lambda i: (0, i)),
        ],
        out_specs=pl.BlockSpec((BLK, D), lambda i: (i, 0)),
        out_shape=jax.ShapeDtypeStruct((NP, D), jnp.float32),
    )(h1, hres, acc, cnt)


def kernel(features, edge_index, W1, b1, W2, b2, W3, b3, W4, b4, Wres, bres):
    n = features.shape[0]
    e = edge_index.shape[1]
    xp = jnp.pad(features, ((0, NP - n), (0, 0)))
    wt = jnp.concatenate([W1.T, W2.T, W3.T, W4.T, Wres.T], axis=1)
    bc = jnp.concatenate([b1, b2, b3, b4, bres]).reshape(1, 5 * D)
    h1, h23, h4, hres = _matmuls(xp, wt, bc)
    src = edge_index[0]
    dst = edge_index[1]
    acc, cnt = _sc_edge_kernel(e)(h23, h4, src, dst)
    z = _combine(h1, hres, acc, cnt.reshape(NW, NP))
    return z[:n]


# speedup vs baseline: 4.8576x; 2.8152x over previous
import jax
import jax.numpy as jnp
from jax import lax
from jax.experimental import pallas as pl
from jax.experimental.pallas import tpu as pltpu
from jax.experimental.pallas import tpu_sc as plsc

D = 128
K = 80
NC = 2
NS = 16
NW = NC * NS
L = 16
NP = 10240
BLK = 1024


def _mm_body(x_ref, w_ref, b_ref, h1_ref, h23_ref, h4_ref, hr_ref):
    h = jnp.dot(x_ref[...], w_ref[...], preferred_element_type=jnp.float32)
    h = h + b_ref[...]
    h1_ref[...] = h[:, 0:D]
    h23_ref[...] = h[:, D:3 * D]
    h4_ref[...] = h[:, 3 * D:4 * D]
    hr_ref[...] = h[:, 4 * D:5 * D]


def _matmuls(x, wt, bc):
    grid = NP // BLK
    return pl.pallas_call(
        _mm_body,
        grid=(grid,),
        in_specs=[
            pl.BlockSpec((BLK, D), lambda i: (i, 0)),
            pl.BlockSpec((D, 5 * D), lambda i: (0, 0)),
            pl.BlockSpec((1, 5 * D), lambda i: (0, 0)),
        ],
        out_specs=[
            pl.BlockSpec((BLK, D), lambda i: (i, 0)),
            pl.BlockSpec((BLK, 2 * D), lambda i: (i, 0)),
            pl.BlockSpec((BLK, D), lambda i: (i, 0)),
            pl.BlockSpec((BLK, D), lambda i: (i, 0)),
        ],
        out_shape=[
            jax.ShapeDtypeStruct((NP, D), jnp.float32),
            jax.ShapeDtypeStruct((NP, 2 * D), jnp.float32),
            jax.ShapeDtypeStruct((NP, D), jnp.float32),
            jax.ShapeDtypeStruct((NP, D), jnp.float32),
        ],
    )(x, wt, bc)


def _sc_edge_kernel(e):
    nchunks = e // K
    per_core = nchunks // NC
    per_sub = per_core // NS
    rps = NP // NS
    mesh = plsc.VectorSubcoreMesh(core_axis_name="c", subcore_axis_name="s")

    def body(h23_hbm, h4_hbm, src_hbm, dst_hbm, acc_out, cnt_out,
             src_v, dst_v, g23, g4, hist, acc_sh, sem23, sem4):
        cid = lax.axis_index("c")
        sid = lax.axis_index("s")
        wid = sid * NC + cid

        zero = jnp.zeros((L,), jnp.float32)

        def zfill(r, _):
            for j in range(D // L):
                g4[r, pl.ds(j * L, L)] = zero
            return 0
        lax.fori_loop(0, K, zfill, 0, unroll=False)

        def hfill(r, _):
            hist[pl.ds(r * L, L)] = zero
            return 0
        lax.fori_loop(0, NP // L, hfill, 0, unroll=False)

        r0 = pl.multiple_of(sid * rps, 8)
        for t in range(rps // K):
            pltpu.sync_copy(g4, acc_sh.at[pl.ds(r0 + t * K, K)])

        plsc.subcore_barrier()

        def chunk_body(i, _):
            c = cid * per_core + sid + i * NS
            base = c * K
            pltpu.sync_copy(src_hbm.at[pl.ds(base, K)], src_v)
            pltpu.sync_copy(dst_hbm.at[pl.ds(base, K)], dst_v)
            cp23 = pltpu.async_copy(h23_hbm.at[src_v], g23, sem23)
            cp4 = pltpu.async_copy(h4_hbm.at[dst_v], g4, sem4)

            for j in range(K // L):
                idxv = dst_v[pl.ds(j * L, L)]
                cnt, lastm = plsc.scan_count(idxv)
                old = plsc.load_gather(hist, [idxv], mask=lastm)
                plsc.store_scatter(
                    hist, [idxv], old + cnt.astype(jnp.float32),
                    mask=lastm)

            cp23.wait()
            cp4.wait()

            def row_body(r, _):
                vals = []
                for j in range(D // L):
                    sl = pl.ds(j * L, L)
                    x2 = g23[r, sl]
                    x3 = g23[r, pl.ds(D + j * L, L)]
                    x4 = g4[r, sl]
                    vals.append(x2 / (1.0 + jnp.exp(-(x3 + x4))))
                for j in range(D // L):
                    g4[r, pl.ds(j * L, L)] = vals[j]
                return 0
            lax.fori_loop(0, K, row_body, 0, unroll=False)

            pltpu.sync_copy(g4, acc_sh.at[dst_v], add=True)
            return 0

        lax.fori_loop(0, per_sub, chunk_body, 0, unroll=False)

        plsc.subcore_barrier()

        for t in range(rps // K):
            row = r0 + t * K
            pltpu.sync_copy(acc_sh.at[pl.ds(row, K)], g4)
            pltpu.sync_copy(g4, acc_out.at[cid, pl.ds(row, K)])

        pltpu.sync_copy(hist, cnt_out.at[pl.ds(wid * NP, NP)])

    return pl.kernel(
        body,
        out_type=[
            jax.ShapeDtypeStruct((NC, NP, D), jnp.float32),
            jax.ShapeDtypeStruct((NW * NP,), jnp.float32),
        ],
        mesh=mesh,
        compiler_params=pltpu.CompilerParams(needs_layout_passes=False),
        scratch_types=[
            pltpu.VMEM((K,), jnp.int32),
            pltpu.VMEM((K,), jnp.int32),
            pltpu.VMEM((K, 2 * D), jnp.float32),
            pltpu.VMEM((K, D), jnp.float32),
            pltpu.VMEM((NP,), jnp.float32),
            pltpu.VMEM_SHARED((NP, D), jnp.float32),
            pltpu.SemaphoreType.DMA,
            pltpu.SemaphoreType.DMA,
        ],
    )


def _comb_body(h1_ref, hr_ref, acc_ref, cnt_ref, o_ref):
    a = acc_ref[0] + acc_ref[1]
    c = jnp.sum(cnt_ref[...], axis=0)[:, None]
    o_ref[...] = (h1_ref[...] + a) / jnp.maximum(c, 1.0) + hr_ref[...]


def _combine(h1, hres, acc, cnt):
    grid = NP // BLK
    return pl.pallas_call(
        _comb_body,
        grid=(grid,),
        in_specs=[
            pl.BlockSpec((BLK, D), lambda i: (i, 0)),
            pl.BlockSpec((BLK, D), lambda i: (i, 0)),
            pl.BlockSpec((NC, BLK, D), lambda i: (0, i, 0)),
            pl.BlockSpec((NW, BLK), lambda i: (0, i)),
        ],
        out_specs=pl.BlockSpec((BLK, D), lambda i: (i, 0)),
        out_shape=jax.ShapeDtypeStruct((NP, D), jnp.float32),
    )(h1, hres, acc, cnt)


def kernel(features, edge_index, W1, b1, W2, b2, W3, b3, W4, b4, Wres, bres):
    n = features.shape[0]
    e = edge_index.shape[1]
    xp = jnp.pad(features, ((0, NP - n), (0, 0)))
    wt = jnp.concatenate([W1.T, W2.T, W3.T, W4.T, Wres.T], axis=1)
    bc = jnp.concatenate([b1, b2, b3, b4, bres]).reshape(1, 5 * D)
    h1, h23, h4, hres = _matmuls(xp, wt, bc)
    src = edge_index[0]
    dst = edge_index[1]
    acc, cnt = _sc_edge_kernel(e)(h23, h4, src, dst)
    z = _combine(h1, hres, acc, cnt.reshape(NW, NP))
    return z[:n]


# speedup vs baseline: 6.5320x; 1.3447x over previous
import jax
import jax.numpy as jnp
from jax import lax
from jax.experimental import pallas as pl
from jax.experimental.pallas import tpu as pltpu
from jax.experimental.pallas import tpu_sc as plsc

D = 128
K = 40
NC = 2
NS = 16
NW = NC * NS
L = 16
NP = 10240
BLK = 1024


def _mm_body(x_ref, w_ref, b_ref, h1_ref, h23_ref, h4_ref, hr_ref):
    h = jnp.dot(x_ref[...], w_ref[...], preferred_element_type=jnp.float32)
    h = h + b_ref[...]
    h1_ref[...] = h[:, 0:D]
    h23_ref[...] = h[:, D:3 * D]
    h4_ref[...] = h[:, 3 * D:4 * D]
    hr_ref[...] = h[:, 4 * D:5 * D]


def _matmuls(x, wt, bc):
    grid = NP // BLK
    return pl.pallas_call(
        _mm_body,
        grid=(grid,),
        in_specs=[
            pl.BlockSpec((BLK, D), lambda i: (i, 0)),
            pl.BlockSpec((D, 5 * D), lambda i: (0, 0)),
            pl.BlockSpec((1, 5 * D), lambda i: (0, 0)),
        ],
        out_specs=[
            pl.BlockSpec((BLK, D), lambda i: (i, 0)),
            pl.BlockSpec((BLK, 2 * D), lambda i: (i, 0)),
            pl.BlockSpec((BLK, D), lambda i: (i, 0)),
            pl.BlockSpec((BLK, D), lambda i: (i, 0)),
        ],
        out_shape=[
            jax.ShapeDtypeStruct((NP, D), jnp.float32),
            jax.ShapeDtypeStruct((NP, 2 * D), jnp.float32),
            jax.ShapeDtypeStruct((NP, D), jnp.float32),
            jax.ShapeDtypeStruct((NP, D), jnp.float32),
        ],
    )(x, wt, bc)


def _sc_edge_kernel(e):
    nchunks = e // K
    per_core = nchunks // NC
    per_sub = per_core // NS
    pairs = per_sub // 2
    rps = NP // NS
    mesh = plsc.VectorSubcoreMesh(core_axis_name="c", subcore_axis_name="s")

    def body(h23_hbm, h4_hbm, src_hbm, dst_hbm, acc_out, cnt_out,
             src_v, dst_v, g23, g4, hist, acc_sh,
             si0, si1, di0, di1, s23a, s23b, s4a, s4b):
        cid = lax.axis_index("c")
        sid = lax.axis_index("s")
        wid = sid * NC + cid

        sis = (si0, si1)
        dis = (di0, di1)
        s23 = (s23a, s23b)
        s4 = (s4a, s4b)

        zero = jnp.zeros((L,), jnp.float32)

        def zfill(r, _):
            for j in range(D // L):
                g4[0, r, pl.ds(j * L, L)] = zero
            return 0
        lax.fori_loop(0, K, zfill, 0, unroll=False)

        def hfill(r, _):
            hist[pl.ds(r * L, L)] = zero
            return 0
        lax.fori_loop(0, NP // L, hfill, 0, unroll=False)

        r0 = pl.multiple_of(sid * rps, 8)
        for t in range(rps // K):
            pltpu.sync_copy(g4.at[0], acc_sh.at[pl.ds(r0 + t * K, K)])

        plsc.subcore_barrier()

        def idx_issue(i, p):
            base = (cid * per_core + sid + i * NS) * K
            pltpu.async_copy(src_hbm.at[pl.ds(base, K)], src_v.at[p], sis[p])
            pltpu.async_copy(dst_hbm.at[pl.ds(base, K)], dst_v.at[p], dis[p])

        def idx_wait(p):
            pltpu.make_async_copy(src_hbm.at[pl.ds(0, K)],
                                  src_v.at[p], sis[p]).wait()
            pltpu.make_async_copy(dst_hbm.at[pl.ds(0, K)],
                                  dst_v.at[p], dis[p]).wait()

        def gather_issue(p):
            pltpu.async_copy(h23_hbm.at[src_v.at[p]], g23.at[p], s23[p])
            pltpu.async_copy(h4_hbm.at[dst_v.at[p]], g4.at[p], s4[p])

        def gather_wait(p):
            pltpu.make_async_copy(h23_hbm.at[src_v.at[p]],
                                  g23.at[p], s23[p]).wait()
            pltpu.make_async_copy(h4_hbm.at[dst_v.at[p]],
                                  g4.at[p], s4[p]).wait()

        lane = jnp.arange(L, dtype=jnp.int32)
        tailm = lane >= (3 * L - K)

        def hist_update(p):
            for off, msk in ((0, None), (L, None), (K - L, tailm)):
                idxv = dst_v[p, pl.ds(off, L)]
                cnt, lastm = plsc.scan_count(idxv, msk)
                old = plsc.load_gather(hist, [idxv], mask=lastm)
                plsc.store_scatter(
                    hist, [idxv], old + cnt.astype(jnp.float32), mask=lastm)

        def compute(p):
            def row_body(r, _):
                vals = []
                for j in range(D // L):
                    sl = pl.ds(j * L, L)
                    x2 = g23[p, r, sl]
                    x3 = g23[p, r, pl.ds(D + j * L, L)]
                    x4 = g4[p, r, sl]
                    vals.append(x2 / (1.0 + jnp.exp(-(x3 + x4))))
                for j in range(D // L):
                    g4[p, r, pl.ds(j * L, L)] = vals[j]
                return 0
            lax.fori_loop(0, K, row_body, 0, unroll=False)

        idx_issue(0, 0)
        idx_issue(1, 1)
        idx_wait(0)
        gather_issue(0)

        def pair_body(i2, _):
            for b in (0, 1):
                i = 2 * i2 + b
                p, q = b, 1 - b

                @pl.when(i < per_sub - 1)
                def _():
                    idx_wait(q)
                    gather_issue(q)

                gather_wait(p)
                hist_update(p)
                compute(p)
                pltpu.sync_copy(g4.at[p], acc_sh.at[dst_v.at[p]], add=True)

                @pl.when(i + 2 < per_sub)
                def _():
                    idx_issue(i + 2, p)
            return 0

        lax.fori_loop(0, pairs, pair_body, 0, unroll=False)

        plsc.subcore_barrier()

        for t in range(rps // K):
            row = r0 + t * K
            pltpu.sync_copy(acc_sh.at[pl.ds(row, K)], g4.at[0])
            pltpu.sync_copy(g4.at[0], acc_out.at[cid, pl.ds(row, K)])

        pltpu.sync_copy(hist, cnt_out.at[pl.ds(wid * NP, NP)])

    return pl.kernel(
        body,
        out_type=[
            jax.ShapeDtypeStruct((NC, NP, D), jnp.float32),
            jax.ShapeDtypeStruct((NW * NP,), jnp.float32),
        ],
        mesh=mesh,
        compiler_params=pltpu.CompilerParams(needs_layout_passes=False),
        scratch_types=[
            pltpu.VMEM((2, K), jnp.int32),
            pltpu.VMEM((2, K), jnp.int32),
            pltpu.VMEM((2, K, 2 * D), jnp.float32),
            pltpu.VMEM((2, K, D), jnp.float32),
            pltpu.VMEM((NP,), jnp.float32),
            pltpu.VMEM_SHARED((NP, D), jnp.float32),
            pltpu.SemaphoreType.DMA,
            pltpu.SemaphoreType.DMA,
            pltpu.SemaphoreType.DMA,
            pltpu.SemaphoreType.DMA,
            pltpu.SemaphoreType.DMA,
            pltpu.SemaphoreType.DMA,
            pltpu.SemaphoreType.DMA,
            pltpu.SemaphoreType.DMA,
        ],
    )


def _comb_body(h1_ref, hr_ref, acc_ref, cnt_ref, o_ref):
    a = acc_ref[0] + acc_ref[1]
    c = jnp.sum(cnt_ref[...], axis=0)[:, None]
    o_ref[...] = (h1_ref[...] + a) / jnp.maximum(c, 1.0) + hr_ref[...]


def _combine(h1, hres, acc, cnt):
    grid = NP // BLK
    return pl.pallas_call(
        _comb_body,
        grid=(grid,),
        in_specs=[
            pl.BlockSpec((BLK, D), lambda i: (i, 0)),
            pl.BlockSpec((BLK, D), lambda i: (i, 0)),
            pl.BlockSpec((NC, BLK, D), lambda i: (0, i, 0)),
            pl.BlockSpec((NW, BLK), lambda i: (0, i)),
        ],
        out_specs=pl.BlockSpec((BLK, D), lambda i: (i, 0)),
        out_shape=jax.ShapeDtypeStruct((NP, D), jnp.float32),
    )(h1, hres, acc, cnt)


def kernel(features, edge_index, W1, b1, W2, b2, W3, b3, W4, b4, Wres, bres):
    n = features.shape[0]
    e = edge_index.shape[1]
    xp = jnp.pad(features, ((0, NP - n), (0, 0)))
    wt = jnp.concatenate([W1.T, W2.T, W3.T, W4.T, Wres.T], axis=1)
    bc = jnp.concatenate([b1, b2, b3, b4, bres]).reshape(1, 5 * D)
    h1, h23, h4, hres = _matmuls(xp, wt, bc)
    src = edge_index[0]
    dst = edge_index[1]
    acc, cnt = _sc_edge_kernel(e)(h23, h4, src, dst)
    z = _combine(h1, hres, acc, cnt.reshape(NW, NP))
    return z[:n]


# speedup vs baseline: 7.3022x; 1.1179x over previous
import jax
import jax.numpy as jnp
from jax import lax
from jax.experimental import pallas as pl
from jax.experimental.pallas import tpu as pltpu
from jax.experimental.pallas import tpu_sc as plsc

D = 128
K = 40
NC = 2
NS = 16
NW = NC * NS
L = 16
NP = 10240
BLK = 1024


def _mm_body(x_ref, w_ref, b_ref, h1_ref, h23_ref, h4_ref, hr_ref):
    h = jnp.dot(x_ref[...], w_ref[...], preferred_element_type=jnp.float32)
    h = h + b_ref[...]
    h1_ref[...] = h[:, 0:D]
    h23_ref[...] = h[:, D:3 * D]
    h4_ref[...] = h[:, 3 * D:4 * D]
    hr_ref[...] = h[:, 4 * D:5 * D]


def _matmuls(x, wt, bc):
    grid = NP // BLK
    return pl.pallas_call(
        _mm_body,
        grid=(grid,),
        in_specs=[
            pl.BlockSpec((BLK, D), lambda i: (i, 0)),
            pl.BlockSpec((D, 5 * D), lambda i: (0, 0)),
            pl.BlockSpec((1, 5 * D), lambda i: (0, 0)),
        ],
        out_specs=[
            pl.BlockSpec((BLK, D), lambda i: (i, 0)),
            pl.BlockSpec((BLK, 2 * D), lambda i: (i, 0)),
            pl.BlockSpec((BLK, D), lambda i: (i, 0)),
            pl.BlockSpec((BLK, D), lambda i: (i, 0)),
        ],
        out_shape=[
            jax.ShapeDtypeStruct((NP, D), jnp.float32),
            jax.ShapeDtypeStruct((NP, 2 * D), jnp.float32),
            jax.ShapeDtypeStruct((NP, D), jnp.float32),
            jax.ShapeDtypeStruct((NP, D), jnp.float32),
        ],
    )(x, wt, bc)


def _sc_edge_kernel(e):
    nchunks = e // K
    per_core = nchunks // NC
    per_sub = per_core // NS
    pairs = per_sub // 2
    rps = NP // NS
    mesh = plsc.VectorSubcoreMesh(core_axis_name="c", subcore_axis_name="s")

    def body(h23_hbm, h4_hbm, src_hbm, dst_hbm, acc_out, cnt_out,
             src_v, dst_v, dstS, g23, g4, hist, acc_sh,
             si0, si1, di0, di1, s23a, s23b, s4a, s4b, ssa, ssb):
        cid = lax.axis_index("c")
        sid = lax.axis_index("s")
        wid = sid * NC + cid

        sis = (si0, si1)
        dis = (di0, di1)
        s23 = (s23a, s23b)
        s4 = (s4a, s4b)
        ssc = (ssa, ssb)

        zero = jnp.zeros((L,), jnp.float32)

        def zfill(r, _):
            for j in range(D // L):
                g4[0, r, pl.ds(j * L, L)] = zero
            return 0
        lax.fori_loop(0, K, zfill, 0, unroll=False)

        def hfill(r, _):
            hist[pl.ds(r * L, L)] = zero
            return 0
        lax.fori_loop(0, NP // L, hfill, 0, unroll=False)

        r0 = pl.multiple_of(sid * rps, 8)
        for t in range(rps // K):
            pltpu.sync_copy(g4.at[0], acc_sh.at[pl.ds(r0 + t * K, K)])

        plsc.subcore_barrier()

        def idx_issue(i, p):
            base = (cid * per_core + sid + i * NS) * K
            pltpu.async_copy(src_hbm.at[pl.ds(base, K)], src_v.at[p], sis[p])
            pltpu.async_copy(dst_hbm.at[pl.ds(base, K)], dst_v.at[p], dis[p])

        def idx_wait(p):
            pltpu.make_async_copy(src_hbm.at[pl.ds(0, K)],
                                  src_v.at[p], sis[p]).wait()
            pltpu.make_async_copy(dst_hbm.at[pl.ds(0, K)],
                                  dst_v.at[p], dis[p]).wait()

        def gather_issue(p):
            pltpu.async_copy(h23_hbm.at[src_v.at[p]], g23.at[p], s23[p])
            pltpu.async_copy(h4_hbm.at[dst_v.at[p]], g4.at[p], s4[p])

        def gather_wait(p):
            pltpu.make_async_copy(h23_hbm.at[src_v.at[p]],
                                  g23.at[p], s23[p]).wait()
            pltpu.make_async_copy(h4_hbm.at[dst_v.at[p]],
                                  g4.at[p], s4[p]).wait()

        lane = jnp.arange(L, dtype=jnp.int32)
        tailm = lane >= (3 * L - K)

        def hist_update(p):
            for off, msk in ((0, None), (L, None), (K - L, tailm)):
                idxv = dst_v[p, pl.ds(off, L)]
                cnt, lastm = plsc.scan_count(idxv, msk)
                old = plsc.load_gather(hist, [idxv], mask=lastm)
                plsc.store_scatter(
                    hist, [idxv], old + cnt.astype(jnp.float32), mask=lastm)

        def compute(p):
            def row_body(r, _):
                vals = []
                for j in range(D // L):
                    sl = pl.ds(j * L, L)
                    x2 = g23[p, r, sl]
                    x3 = g23[p, r, pl.ds(D + j * L, L)]
                    x4 = g4[p, r, sl]
                    vals.append(x2 / (1.0 + jnp.exp(-(x3 + x4))))
                for j in range(D // L):
                    g4[p, r, pl.ds(j * L, L)] = vals[j]
                return 0
            lax.fori_loop(0, K, row_body, 0, unroll=False)

        idx_issue(0, 0)
        idx_issue(1, 1)
        idx_wait(0)
        gather_issue(0)

        def scatter_wait(p):
            pltpu.make_async_copy(g4.at[p],
                                  acc_sh.at[dstS.at[p]], ssc[p]).wait()

        def pair_body(i2, _):
            for b in (0, 1):
                i = 2 * i2 + b
                p, q = b, 1 - b

                @pl.when(i < per_sub - 1)
                def _():
                    idx_wait(q)

                @pl.when(i >= 1)
                def _():
                    scatter_wait(q)

                @pl.when(i < per_sub - 1)
                def _():
                    gather_issue(q)

                hist_update(p)
                gather_wait(p)
                for off in (0, L, K - L):
                    dstS[p, pl.ds(off, L)] = dst_v[p, pl.ds(off, L)]
                compute(p)
                pltpu.async_copy(g4.at[p], acc_sh.at[dstS.at[p]],
                                 ssc[p], add=True)

                @pl.when(i + 2 < per_sub)
                def _():
                    idx_issue(i + 2, p)
            return 0

        lax.fori_loop(0, pairs, pair_body, 0, unroll=False)

        scatter_wait(1)
        plsc.subcore_barrier()

        for t in range(rps // K):
            row = r0 + t * K
            pltpu.sync_copy(acc_sh.at[pl.ds(row, K)], g4.at[0])
            pltpu.sync_copy(g4.at[0], acc_out.at[cid, pl.ds(row, K)])

        pltpu.sync_copy(hist, cnt_out.at[pl.ds(wid * NP, NP)])

    return pl.kernel(
        body,
        out_type=[
            jax.ShapeDtypeStruct((NC, NP, D), jnp.float32),
            jax.ShapeDtypeStruct((NW * NP,), jnp.float32),
        ],
        mesh=mesh,
        compiler_params=pltpu.CompilerParams(needs_layout_passes=False),
        scratch_types=[
            pltpu.VMEM((2, K), jnp.int32),
            pltpu.VMEM((2, K), jnp.int32),
            pltpu.VMEM((2, K), jnp.int32),
            pltpu.VMEM((2, K, 2 * D), jnp.float32),
            pltpu.VMEM((2, K, D), jnp.float32),
            pltpu.VMEM((NP,), jnp.float32),
            pltpu.VMEM_SHARED((NP, D), jnp.float32),
            pltpu.SemaphoreType.DMA,
            pltpu.SemaphoreType.DMA,
            pltpu.SemaphoreType.DMA,
            pltpu.SemaphoreType.DMA,
            pltpu.SemaphoreType.DMA,
            pltpu.SemaphoreType.DMA,
            pltpu.SemaphoreType.DMA,
            pltpu.SemaphoreType.DMA,
            pltpu.SemaphoreType.DMA,
            pltpu.SemaphoreType.DMA,
        ],
    )


def _comb_body(h1_ref, hr_ref, acc_ref, cnt_ref, o_ref):
    a = acc_ref[0] + acc_ref[1]
    c = jnp.sum(cnt_ref[...], axis=0)[:, None]
    o_ref[...] = (h1_ref[...] + a) / jnp.maximum(c, 1.0) + hr_ref[...]


def _combine(h1, hres, acc, cnt):
    grid = NP // BLK
    return pl.pallas_call(
        _comb_body,
        grid=(grid,),
        in_specs=[
            pl.BlockSpec((BLK, D), lambda i: (i, 0)),
            pl.BlockSpec((BLK, D), lambda i: (i, 0)),
            pl.BlockSpec((NC, BLK, D), lambda i: (0, i, 0)),
            pl.BlockSpec((NW, BLK), lambda i: (0, i)),
        ],
        out_specs=pl.BlockSpec((BLK, D), lambda i: (i, 0)),
        out_shape=jax.ShapeDtypeStruct((NP, D), jnp.float32),
    )(h1, hres, acc, cnt)


def kernel(features, edge_index, W1, b1, W2, b2, W3, b3, W4, b4, Wres, bres):
    n = features.shape[0]
    e = edge_index.shape[1]
    xp = jnp.pad(features, ((0, NP - n), (0, 0)))
    wt = jnp.concatenate([W1.T, W2.T, W3.T, W4.T, Wres.T], axis=1)
    bc = jnp.concatenate([b1, b2, b3, b4, bres]).reshape(1, 5 * D)
    h1, h23, h4, hres = _matmuls(xp, wt, bc)
    src = edge_index[0]
    dst = edge_index[1]
    acc, cnt = _sc_edge_kernel(e)(h23, h4, src, dst)
    z = _combine(h1, hres, acc, cnt.reshape(NW, NP))
    return z[:n]


# speedup vs baseline: 7.3410x; 1.0053x over previous
import jax
import jax.numpy as jnp
from jax import lax
from jax.experimental import pallas as pl
from jax.experimental.pallas import tpu as pltpu
from jax.experimental.pallas import tpu_sc as plsc

D = 128
K = 40
NC = 2
NS = 16
NW = NC * NS
L = 16
NP = 10240
BLK = 1024


def _mm_body(x_ref, w_ref, b_ref, h1_ref, h23_ref, h4_ref, hr_ref):
    h = jnp.dot(x_ref[...], w_ref[...], preferred_element_type=jnp.float32)
    h = h + b_ref[...]
    h1_ref[...] = h[:, 0:D]
    h23_ref[...] = h[:, D:3 * D]
    h4_ref[...] = h[:, 3 * D:4 * D]
    hr_ref[...] = h[:, 4 * D:5 * D]


def _matmuls(x, wt, bc):
    grid = NP // BLK
    return pl.pallas_call(
        _mm_body,
        grid=(grid,),
        in_specs=[
            pl.BlockSpec((BLK, D), lambda i: (i, 0)),
            pl.BlockSpec((D, 5 * D), lambda i: (0, 0)),
            pl.BlockSpec((1, 5 * D), lambda i: (0, 0)),
        ],
        out_specs=[
            pl.BlockSpec((BLK, D), lambda i: (i, 0)),
            pl.BlockSpec((BLK, 2 * D), lambda i: (i, 0)),
            pl.BlockSpec((BLK, D), lambda i: (i, 0)),
            pl.BlockSpec((BLK, D), lambda i: (i, 0)),
        ],
        out_shape=[
            jax.ShapeDtypeStruct((NP, D), jnp.float32),
            jax.ShapeDtypeStruct((NP, 2 * D), jnp.float32),
            jax.ShapeDtypeStruct((NP, D), jnp.float32),
            jax.ShapeDtypeStruct((NP, D), jnp.float32),
        ],
    )(x, wt, bc)


def _sc_edge_kernel(e):
    nchunks = e // K
    per_core = nchunks // NC
    per_sub = per_core // NS
    pairs = per_sub // 2
    rps = NP // NS
    mesh = plsc.VectorSubcoreMesh(core_axis_name="c", subcore_axis_name="s")

    def body(h23_hbm, h4_hbm, src_hbm, dst_hbm, acc_out, cnt_out,
             src_v, dst_v, dstS, g23, g4, hist, acc_sh,
             si0, si1, di0, di1, s23a, s23b, s4a, s4b, ssa, ssb):
        cid = lax.axis_index("c")
        sid = lax.axis_index("s")
        wid = sid * NC + cid

        sis = (si0, si1)
        dis = (di0, di1)
        s23 = (s23a, s23b)
        s4 = (s4a, s4b)
        ssc = (ssa, ssb)

        zero = jnp.zeros((L,), jnp.float32)

        def zfill(r, _):
            for j in range(D // L):
                g4[0, r, pl.ds(j * L, L)] = zero
            return 0
        lax.fori_loop(0, K, zfill, 0, unroll=False)

        def hfill(r, _):
            hist[pl.ds(r * L, L)] = zero
            return 0
        lax.fori_loop(0, NP // L, hfill, 0, unroll=False)

        r0 = pl.multiple_of(sid * rps, 8)
        for t in range(rps // K):
            pltpu.sync_copy(g4.at[0], acc_sh.at[pl.ds(r0 + t * K, K)])

        plsc.subcore_barrier()

        def idx_issue(i, p):
            base = (cid * per_core + sid + i * NS) * K
            pltpu.async_copy(src_hbm.at[pl.ds(base, K)], src_v.at[p], sis[p])
            pltpu.async_copy(dst_hbm.at[pl.ds(base, K)], dst_v.at[p], dis[p])

        def idx_wait(p):
            pltpu.make_async_copy(src_hbm.at[pl.ds(0, K)],
                                  src_v.at[p], sis[p]).wait()
            pltpu.make_async_copy(dst_hbm.at[pl.ds(0, K)],
                                  dst_v.at[p], dis[p]).wait()

        def gather_issue(p):
            pltpu.async_copy(h23_hbm.at[src_v.at[p]], g23.at[p], s23[p])
            pltpu.async_copy(h4_hbm.at[dst_v.at[p]], g4.at[p], s4[p])

        def gather_wait(p):
            pltpu.make_async_copy(h23_hbm.at[src_v.at[p]],
                                  g23.at[p], s23[p]).wait()
            pltpu.make_async_copy(h4_hbm.at[dst_v.at[p]],
                                  g4.at[p], s4[p]).wait()

        lane = jnp.arange(L, dtype=jnp.int32)
        tailm = lane >= (3 * L - K)

        def hist_update(p):
            for off, msk in ((0, None), (L, None), (K - L, tailm)):
                idxv = dst_v[p, pl.ds(off, L)]
                cnt, lastm = plsc.scan_count(idxv, msk)
                old = plsc.load_gather(hist, [idxv], mask=lastm)
                plsc.store_scatter(
                    hist, [idxv], old + cnt.astype(jnp.float32), mask=lastm)

        def compute(p):
            def row_body(r, _):
                vals = []
                for j in range(D // L):
                    sl = pl.ds(j * L, L)
                    x2 = g23[p, r, sl]
                    x3 = g23[p, r, pl.ds(D + j * L, L)]
                    x4 = g4[p, r, sl]
                    vals.append(x2 / (1.0 + jnp.exp(-(x3 + x4))))
                for j in range(D // L):
                    g4[p, r, pl.ds(j * L, L)] = vals[j]
                return 0
            lax.fori_loop(0, K, row_body, 0, unroll=2)

        idx_issue(0, 0)
        idx_issue(1, 1)
        idx_wait(0)
        gather_issue(0)

        def scatter_wait(p):
            pltpu.make_async_copy(g4.at[p],
                                  acc_sh.at[dstS.at[p]], ssc[p]).wait()

        def pair_body(i2, _):
            for b in (0, 1):
                i = 2 * i2 + b
                p, q = b, 1 - b

                @pl.when(i < per_sub - 1)
                def _():
                    idx_wait(q)

                @pl.when(i >= 1)
                def _():
                    scatter_wait(q)

                @pl.when(i < per_sub - 1)
                def _():
                    gather_issue(q)

                hist_update(p)
                gather_wait(p)
                for off in (0, L, K - L):
                    dstS[p, pl.ds(off, L)] = dst_v[p, pl.ds(off, L)]
                compute(p)
                pltpu.async_copy(g4.at[p], acc_sh.at[dstS.at[p]],
                                 ssc[p], add=True)

                @pl.when(i + 2 < per_sub)
                def _():
                    idx_issue(i + 2, p)
            return 0

        lax.fori_loop(0, pairs, pair_body, 0, unroll=False)

        scatter_wait(1)
        plsc.subcore_barrier()

        for t in range(rps // K):
            row = r0 + t * K
            pltpu.sync_copy(acc_sh.at[pl.ds(row, K)], g4.at[0])
            pltpu.sync_copy(g4.at[0], acc_out.at[cid, pl.ds(row, K)])

        pltpu.sync_copy(hist, cnt_out.at[pl.ds(wid * NP, NP)])

    return pl.kernel(
        body,
        out_type=[
            jax.ShapeDtypeStruct((NC, NP, D), jnp.float32),
            jax.ShapeDtypeStruct((NW * NP,), jnp.float32),
        ],
        mesh=mesh,
        compiler_params=pltpu.CompilerParams(needs_layout_passes=False),
        scratch_types=[
            pltpu.VMEM((2, K), jnp.int32),
            pltpu.VMEM((2, K), jnp.int32),
            pltpu.VMEM((2, K), jnp.int32),
            pltpu.VMEM((2, K, 2 * D), jnp.float32),
            pltpu.VMEM((2, K, D), jnp.float32),
            pltpu.VMEM((NP,), jnp.float32),
            pltpu.VMEM_SHARED((NP, D), jnp.float32),
            pltpu.SemaphoreType.DMA,
            pltpu.SemaphoreType.DMA,
            pltpu.SemaphoreType.DMA,
            pltpu.SemaphoreType.DMA,
            pltpu.SemaphoreType.DMA,
            pltpu.SemaphoreType.DMA,
            pltpu.SemaphoreType.DMA,
            pltpu.SemaphoreType.DMA,
            pltpu.SemaphoreType.DMA,
            pltpu.SemaphoreType.DMA,
        ],
    )


def _comb_body(h1_ref, hr_ref, acc_ref, cnt_ref, o_ref):
    a = acc_ref[0] + acc_ref[1]
    c = jnp.sum(cnt_ref[...], axis=0)[:, None]
    o_ref[...] = (h1_ref[...] + a) / jnp.maximum(c, 1.0) + hr_ref[...]


def _combine(h1, hres, acc, cnt):
    grid = NP // BLK
    return pl.pallas_call(
        _comb_body,
        grid=(grid,),
        in_specs=[
            pl.BlockSpec((BLK, D), lambda i: (i, 0)),
            pl.BlockSpec((BLK, D), lambda i: (i, 0)),
            pl.BlockSpec((NC, BLK, D), lambda i: (0, i, 0)),
            pl.BlockSpec((NW, BLK), lambda i: (0, i)),
        ],
        out_specs=pl.BlockSpec((BLK, D), lambda i: (i, 0)),
        out_shape=jax.ShapeDtypeStruct((NP, D), jnp.float32),
    )(h1, hres, acc, cnt)


def kernel(features, edge_index, W1, b1, W2, b2, W3, b3, W4, b4, Wres, bres):
    n = features.shape[0]
    e = edge_index.shape[1]
    xp = jnp.pad(features, ((0, NP - n), (0, 0)))
    wt = jnp.concatenate([W1.T, W2.T, W3.T, W4.T, Wres.T], axis=1)
    bc = jnp.concatenate([b1, b2, b3, b4, bres]).reshape(1, 5 * D)
    h1, h23, h4, hres = _matmuls(xp, wt, bc)
    src = edge_index[0]
    dst = edge_index[1]
    acc, cnt = _sc_edge_kernel(e)(h23, h4, src, dst)
    z = _combine(h1, hres, acc, cnt.reshape(NW, NP))
    return z[:n]
